# baseline (speedup 1.0000x reference)
import functools

import jax
import jax.numpy as jnp
from jax import lax
from jax.experimental import pallas as pl
from jax.experimental.pallas import tpu as pltpu
from jax.experimental.pallas import tpu_sc as plsc

_DBLK = 4096
_D_SC = 4096
_BCH = 8


def _tc_body(x_ref, pos_ref, lw_ref, o_ref, *, P, levels):
    xr = x_ref[0, 0, :]
    idx = jnp.clip(jnp.round(xr * (levels - 1)), 0, levels - 1).astype(jnp.int32)
    ii = jax.lax.broadcasted_iota(jnp.int32, (levels, P), 0)
    at = (ii == idx[None, :]).astype(jnp.bfloat16)
    g = jnp.dot(at, pos_ref[...], preferred_element_type=jnp.float32)
    s = jnp.sum(g * lw_ref[...], axis=0)
    o_ref[0, 0, :] = jnp.where(s > 0, 1.0, -1.0).astype(jnp.float32)


def _tc_part(x, position, level_weight):
    B = x.shape[0]
    P = x.shape[1] * x.shape[2]
    D = position.shape[1]
    levels = level_weight.shape[0]
    x_flat = x.reshape(B, 1, P)
    pos_bf = position.astype(jnp.bfloat16)
    dblk = min(_DBLK, D)
    out = pl.pallas_call(
        functools.partial(_tc_body, P=P, levels=levels),
        grid=(D // dblk, B),
        in_specs=[
            pl.BlockSpec((1, 1, P), lambda j, b: (b, 0, 0)),
            pl.BlockSpec((P, dblk), lambda j, b: (0, j)),
            pl.BlockSpec((levels, dblk), lambda j, b: (0, j)),
        ],
        out_specs=pl.BlockSpec((1, 1, dblk), lambda j, b: (b, 0, j)),
        out_shape=jax.ShapeDtypeStruct((B, 1, D), jnp.float32),
        compiler_params=pltpu.CompilerParams(
            dimension_semantics=("arbitrary", "arbitrary"),
        ),
    )(x_flat, pos_bf, level_weight)
    return out.reshape(B, D)


def _sc_part(idx, position, lvl, base, B, P, D):
    NW = 32
    DT = D // NW
    NK = DT // 16
    mesh = plsc.VectorSubcoreMesh(core_axis_name="c", subcore_axis_name="s")

    @functools.partial(
        pl.kernel,
        mesh=mesh,
        out_type=jax.ShapeDtypeStruct((B, D), jnp.float32),
        scratch_types=[
            pltpu.VMEM((P, DT), jnp.float32),
            pltpu.VMEM((_BCH, P), jnp.int32),
            pltpu.VMEM((B, DT), jnp.float32),
            pltpu.VMEM((DT,), jnp.int32),
            pltpu.VMEM((DT,), jnp.float32),
        ],
    )
    def k(idx_hbm, pos_hbm, lvl_hbm, base_hbm, out_hbm,
          pos_v, idx_v, outf_v, lvl_v, base_v):
        wid = lax.axis_index("s") * 2 + lax.axis_index("c")
        dbase = wid * DT
        pltpu.sync_copy(pos_hbm.at[:, pl.ds(dbase, DT)], pos_v)
        pltpu.sync_copy(lvl_hbm.at[pl.ds(dbase, DT)], lvl_v)
        pltpu.sync_copy(base_hbm.at[pl.ds(dbase, DT)], base_v)

        lvls = [lvl_v[pl.ds(16 * kk, 16)] for kk in range(NK)]
        bases = [base_v[pl.ds(16 * kk, 16)] for kk in range(NK)]

        def s_body(p, accs):
            return tuple(
                accs[kk] + pos_v[p, pl.ds(16 * kk, 16)]
                for kk in range(NK)
            )
        svecs = lax.fori_loop(
            0, P, s_body,
            tuple(jnp.zeros((16,), jnp.float32) for _ in range(NK)),
            unroll=2,
        )

        def b_chunk(bc, _):
            pltpu.sync_copy(idx_hbm.at[pl.ds(bc * _BCH, _BCH), :], idx_v)

            def b_body(bb, _):
                def p_body(pc, accs):
                    tv16 = idx_v[bb, pl.ds(pc * 16, 16)]
                    accs = list(accs)
                    for j in range(16):
                        tv = jnp.full((16,), tv16[j], jnp.int32)
                        p = pc * 16 + j
                        for kk in range(NK):
                            accs[kk] = accs[kk] + jnp.where(
                                tv >= lvls[kk],
                                pos_v[p, pl.ds(16 * kk, 16)],
                                jnp.zeros((16,), jnp.float32),
                            )
                    return tuple(accs)
                fvecs = lax.fori_loop(
                    0, P // 16, p_body,
                    tuple(jnp.zeros((16,), jnp.float32) for _ in range(NK)),
                )
                b = bc * _BCH + bb
                for kk in range(NK):
                    v = bases[kk] * (svecs[kk] - 2.0 * fvecs[kk])
                    outf_v[b, pl.ds(16 * kk, 16)] = jnp.where(
                        v > 0,
                        jnp.full((16,), 1.0, jnp.float32),
                        jnp.full((16,), -1.0, jnp.float32),
                    )
                return 0

            return lax.fori_loop(0, _BCH, b_body, 0)

        lax.fori_loop(0, B // _BCH, b_chunk, 0)
        pltpu.sync_copy(outf_v, out_hbm.at[:, pl.ds(dbase, DT)])

    return k(idx, position, lvl, base)


@jax.jit
def kernel(x, position, level_weight):
    B = x.shape[0]
    P = x.shape[1] * x.shape[2]
    D = position.shape[1]
    levels = level_weight.shape[0]

    d_sc = min(_D_SC, D)
    if d_sc == 0:
        return _tc_part(x, position, level_weight)

    flat = x.reshape(B, P)
    idx = jnp.clip(jnp.round(flat * (levels - 1)), 0, levels - 1).astype(jnp.int32)
    base = level_weight[0]
    lvl = jnp.sum((level_weight == base[None, :]).astype(jnp.int32), axis=0)

    sc_out = _sc_part(
        idx, position[:, :d_sc], lvl[:d_sc], base[:d_sc], B, P, d_sc
    )
    if d_sc == D:
        return sc_out
    tc_out = _tc_part(x, position[:, d_sc:], level_weight[:, d_sc:])
    return jnp.concatenate([sc_out, tc_out], axis=1)


# speedup vs baseline: 2.8522x; 2.8522x over previous
import functools

import jax
import jax.numpy as jnp
from jax import lax
from jax.experimental import pallas as pl
from jax.experimental.pallas import tpu as pltpu
from jax.experimental.pallas import tpu_sc as plsc

_DBLK = 4096
_D_SC = 1024
_BCH = 8


def _tc_body(x_ref, pos_ref, lw_ref, o_ref, *, P, levels):
    xr = x_ref[0, 0, :]
    idx = jnp.clip(jnp.round(xr * (levels - 1)), 0, levels - 1).astype(jnp.int32)
    ii = jax.lax.broadcasted_iota(jnp.int32, (levels, P), 0)
    at = (ii == idx[None, :]).astype(jnp.bfloat16)
    g = jnp.dot(at, pos_ref[...], preferred_element_type=jnp.float32)
    s = jnp.sum(g * lw_ref[...], axis=0)
    o_ref[0, 0, :] = jnp.where(s > 0, 1.0, -1.0).astype(jnp.float32)


def _tc_part(x, position, level_weight):
    B = x.shape[0]
    P = x.shape[1] * x.shape[2]
    D = position.shape[1]
    levels = level_weight.shape[0]
    x_flat = x.reshape(B, 1, P)
    pos_bf = position.astype(jnp.bfloat16)
    dblk = min(_DBLK, D)
    out = pl.pallas_call(
        functools.partial(_tc_body, P=P, levels=levels),
        grid=(D // dblk, B),
        in_specs=[
            pl.BlockSpec((1, 1, P), lambda j, b: (b, 0, 0)),
            pl.BlockSpec((P, dblk), lambda j, b: (0, j)),
            pl.BlockSpec((levels, dblk), lambda j, b: (0, j)),
        ],
        out_specs=pl.BlockSpec((1, 1, dblk), lambda j, b: (b, 0, j)),
        out_shape=jax.ShapeDtypeStruct((B, 1, D), jnp.float32),
        compiler_params=pltpu.CompilerParams(
            dimension_semantics=("arbitrary", "arbitrary"),
        ),
    )(x_flat, pos_bf, level_weight)
    return out.reshape(B, D)


def _sc_part(idx, position, lvl, base, B, P, D):
    NW = 32
    DT = 128
    NK = DT // 16
    ND = D // DT
    NB = NW // ND
    B_t = B // NB
    mesh = plsc.VectorSubcoreMesh(core_axis_name="c", subcore_axis_name="s")

    @functools.partial(
        pl.kernel,
        mesh=mesh,
        out_type=jax.ShapeDtypeStruct((B, D), jnp.float32),
        scratch_types=[
            pltpu.VMEM((P, DT), jnp.float32),
            pltpu.VMEM((_BCH, P), jnp.int32),
            pltpu.VMEM((B_t, DT), jnp.float32),
            pltpu.VMEM((DT,), jnp.int32),
            pltpu.VMEM((DT,), jnp.float32),
            pltpu.VMEM((DT,), jnp.float32),
        ],
    )
    def k(idx_hbm, pos_hbm, lvl_hbm, base_hbm, out_hbm,
          pos_v, idx_v, outf_v, lvl_v, base_v, s_v):
        wid = lax.axis_index("s") * 2 + lax.axis_index("c")
        wd = lax.rem(wid, ND)
        wb = wid // ND
        dbase = wd * DT
        bbase = wb * B_t
        pltpu.sync_copy(pos_hbm.at[:, pl.ds(dbase, DT)], pos_v)
        pltpu.sync_copy(lvl_hbm.at[pl.ds(dbase, DT)], lvl_v)
        pltpu.sync_copy(base_hbm.at[pl.ds(dbase, DT)], base_v)

        lvls = [lvl_v[pl.ds(16 * kk, 16)] for kk in range(NK)]
        zero = jnp.zeros((16,), jnp.float32)
        one = jnp.full((16,), 1.0, jnp.float32)
        mone = jnp.full((16,), -1.0, jnp.float32)

        def s_body(p, accs):
            return tuple(
                accs[kk] + pos_v[p, pl.ds(16 * kk, 16)]
                for kk in range(NK)
            )
        svecs = lax.fori_loop(
            0, P, s_body, tuple(zero for _ in range(NK)), unroll=2,
        )
        for kk in range(NK):
            s_v[pl.ds(16 * kk, 16)] = svecs[kk]

        GB = 2

        def b_chunk(bc, _):
            pltpu.sync_copy(
                idx_hbm.at[pl.ds(bbase + bc * _BCH, _BCH), :], idx_v)

            def b_group(bg, _):
                def p_body(pc, accs):
                    tvs = [idx_v[bg * GB + g, pl.ds(pc * 16, 16)]
                           for g in range(GB)]
                    accs = list(accs)
                    for j in range(16):
                        p = pc * 16 + j
                        pks = [pos_v[p, pl.ds(16 * kk, 16)]
                               for kk in range(NK)]
                        for g in range(GB):
                            tv = jnp.full((16,), tvs[g][j], jnp.int32)
                            for kk in range(NK):
                                i = g * NK + kk
                                accs[i] = accs[i] + jnp.where(
                                    tv >= lvls[kk], pks[kk], zero)
                    return tuple(accs)

                fvecs = lax.fori_loop(
                    0, P // 16, p_body,
                    tuple(zero for _ in range(GB * NK)),
                )
                for g in range(GB):
                    b = bc * _BCH + bg * GB + g
                    for kk in range(NK):
                        sv = s_v[pl.ds(16 * kk, 16)]
                        bv = base_v[pl.ds(16 * kk, 16)]
                        v = bv * (sv - 2.0 * fvecs[g * NK + kk])
                        outf_v[b, pl.ds(16 * kk, 16)] = jnp.where(
                            v > 0, one, mone)
                return 0

            return lax.fori_loop(0, _BCH // GB, b_group, 0)

        lax.fori_loop(0, B_t // _BCH, b_chunk, 0)
        pltpu.sync_copy(
            outf_v, out_hbm.at[pl.ds(bbase, B_t), pl.ds(dbase, DT)])

    return k(idx, position, lvl, base)


@jax.jit
def kernel(x, position, level_weight):
    B = x.shape[0]
    P = x.shape[1] * x.shape[2]
    D = position.shape[1]
    levels = level_weight.shape[0]

    d_sc = min(_D_SC, D)
    if d_sc == 0:
        return _tc_part(x, position, level_weight)

    flat = x.reshape(B, P)
    idx = jnp.clip(jnp.round(flat * (levels - 1)), 0, levels - 1).astype(jnp.int32)
    base = level_weight[0]
    lvl = jnp.sum((level_weight == base[None, :]).astype(jnp.int32), axis=0)

    sc_out = _sc_part(
        idx, position[:, :d_sc], lvl[:d_sc], base[:d_sc], B, P, d_sc
    )
    if d_sc == D:
        return sc_out
    tc_out = _tc_part(x, position[:, d_sc:], level_weight[:, d_sc:])
    return jnp.concatenate([sc_out, tc_out], axis=1)


# speedup vs baseline: 8.9778x; 3.1477x over previous
import functools

import jax
import jax.numpy as jnp
from jax import lax
from jax.experimental import pallas as pl
from jax.experimental.pallas import tpu as pltpu
from jax.experimental.pallas import tpu_sc as plsc

_DBLK = 4096
_D_SC = 256
_BCH = 8


def _tc_body(x_ref, pos_ref, lw_ref, o_ref, *, P, levels):
    xr = x_ref[0, 0, :]
    idx = jnp.clip(jnp.round(xr * (levels - 1)), 0, levels - 1).astype(jnp.int32)
    ii = jax.lax.broadcasted_iota(jnp.int32, (levels, P), 0)
    at = (ii == idx[None, :]).astype(jnp.bfloat16)
    g = jnp.dot(at, pos_ref[...], preferred_element_type=jnp.float32)
    s = jnp.sum(g * lw_ref[...], axis=0)
    o_ref[0, 0, :] = jnp.where(s > 0, 1.0, -1.0).astype(jnp.float32)


def _tc_part(x, position, level_weight):
    B = x.shape[0]
    P = x.shape[1] * x.shape[2]
    D = position.shape[1]
    levels = level_weight.shape[0]
    x_flat = x.reshape(B, 1, P)
    pos_bf = position.astype(jnp.bfloat16)
    dblk = min(_DBLK, D)
    out = pl.pallas_call(
        functools.partial(_tc_body, P=P, levels=levels),
        grid=(D // dblk, B),
        in_specs=[
            pl.BlockSpec((1, 1, P), lambda j, b: (b, 0, 0)),
            pl.BlockSpec((P, dblk), lambda j, b: (0, j)),
            pl.BlockSpec((levels, dblk), lambda j, b: (0, j)),
        ],
        out_specs=pl.BlockSpec((1, 1, dblk), lambda j, b: (b, 0, j)),
        out_shape=jax.ShapeDtypeStruct((B, 1, D), jnp.float32),
        compiler_params=pltpu.CompilerParams(
            dimension_semantics=("arbitrary", "arbitrary"),
        ),
    )(x_flat, pos_bf, level_weight)
    return out.reshape(B, D)


def _sc_part(idx, position, lvl, base, B, P, D):
    NW = 32
    DT = 128
    NK = DT // 16
    ND = D // DT
    NB = NW // ND
    B_t = B // NB
    mesh = plsc.VectorSubcoreMesh(core_axis_name="c", subcore_axis_name="s")

    @functools.partial(
        pl.kernel,
        mesh=mesh,
        out_type=jax.ShapeDtypeStruct((B, D), jnp.float32),
        scratch_types=[
            pltpu.VMEM((P, DT), jnp.float32),
            pltpu.VMEM((_BCH, P), jnp.int32),
            pltpu.VMEM((B_t, DT), jnp.float32),
            pltpu.VMEM((DT,), jnp.int32),
            pltpu.VMEM((DT,), jnp.float32),
            pltpu.VMEM((DT,), jnp.float32),
        ],
    )
    def k(idx_hbm, pos_hbm, lvl_hbm, base_hbm, out_hbm,
          pos_v, idx_v, outf_v, lvl_v, base_v, s_v):
        wid = lax.axis_index("s") * 2 + lax.axis_index("c")
        wd = lax.rem(wid, ND)
        wb = wid // ND
        dbase = wd * DT
        bbase = wb * B_t
        pltpu.sync_copy(pos_hbm.at[:, pl.ds(dbase, DT)], pos_v)
        pltpu.sync_copy(lvl_hbm.at[pl.ds(dbase, DT)], lvl_v)
        pltpu.sync_copy(base_hbm.at[pl.ds(dbase, DT)], base_v)

        lvls = [lvl_v[pl.ds(16 * kk, 16)] for kk in range(NK)]
        zero = jnp.zeros((16,), jnp.float32)
        one = jnp.full((16,), 1.0, jnp.float32)
        mone = jnp.full((16,), -1.0, jnp.float32)

        def s_body(p, accs):
            return tuple(
                accs[kk] + pos_v[p, pl.ds(16 * kk, 16)]
                for kk in range(NK)
            )
        svecs = lax.fori_loop(
            0, P, s_body, tuple(zero for _ in range(NK)), unroll=2,
        )
        for kk in range(NK):
            s_v[pl.ds(16 * kk, 16)] = svecs[kk]

        GB = 2
        lm1s = [lvls[kk] - 1 for kk in range(NK)]

        def b_chunk(bc, _):
            pltpu.sync_copy(
                idx_hbm.at[pl.ds(bbase + bc * _BCH, _BCH), :], idx_v)

            def b_group(bg, _):
                def p_body(pc, accs):
                    tv16s = [idx_v[bg * GB + g, pl.ds(pc * 16, 16)]
                             for g in range(GB)]
                    accs = list(accs)
                    for j in range(16):
                        p = pc * 16 + j
                        jv = jnp.full((16,), j, jnp.int32)
                        pks = [pos_v[p, pl.ds(16 * kk, 16)]
                               for kk in range(NK)]
                        for g in range(GB):
                            tv = tv16s[g].at[jv].get(
                                mode="promise_in_bounds")
                            for kk in range(NK):
                                i = g * NK + kk
                                accs[i] = accs[i] + jnp.where(
                                    tv >= lvls[kk], pks[kk], zero)
                    return tuple(accs)

                fvecs = lax.fori_loop(
                    0, P // 16, p_body,
                    tuple(zero for _ in range(GB * NK)),
                )
                for g in range(GB):
                    b = bc * _BCH + bg * GB + g
                    for kk in range(NK):
                        sv = s_v[pl.ds(16 * kk, 16)]
                        bv = base_v[pl.ds(16 * kk, 16)]
                        v = bv * (sv - 2.0 * fvecs[g * NK + kk])
                        outf_v[b, pl.ds(16 * kk, 16)] = jnp.where(
                            v > 0, one, mone)
                return 0

            return lax.fori_loop(0, _BCH // GB, b_group, 0)

        lax.fori_loop(0, B_t // _BCH, b_chunk, 0)
        pltpu.sync_copy(
            outf_v, out_hbm.at[pl.ds(bbase, B_t), pl.ds(dbase, DT)])

    return k(idx, position, lvl, base)


@jax.jit
def kernel(x, position, level_weight):
    B = x.shape[0]
    P = x.shape[1] * x.shape[2]
    D = position.shape[1]
    levels = level_weight.shape[0]

    d_sc = min(_D_SC, D)
    if d_sc == 0:
        return _tc_part(x, position, level_weight)

    flat = x.reshape(B, P)
    idx = jnp.clip(jnp.round(flat * (levels - 1)), 0, levels - 1).astype(jnp.int32)
    base = level_weight[0]
    lvl = jnp.sum((level_weight == base[None, :]).astype(jnp.int32), axis=0)

    sc_out = _sc_part(
        idx, position[:, :d_sc], lvl[:d_sc], base[:d_sc], B, P, d_sc
    )
    if d_sc == D:
        return sc_out
    tc_out = _tc_part(x, position[:, d_sc:], level_weight[:, d_sc:])
    return jnp.concatenate([sc_out, tc_out], axis=1)


# speedup vs baseline: 9.6036x; 1.0697x over previous
import functools

import jax
import jax.numpy as jnp
from jax import lax
from jax.experimental import pallas as pl
from jax.experimental.pallas import tpu as pltpu
from jax.experimental.pallas import tpu_sc as plsc

_DBLK = 4096
_D_SC = 0
_BCH = 8


_GI = 2


def _tc_body(x_ref, pos_ref, lw_ref, o_ref, *, P, levels):
    ii = jax.lax.broadcasted_iota(jnp.int32, (levels, P), 0)
    ats = []
    for g in range(_GI):
        xr = x_ref[0, g, :]
        idx = jnp.clip(jnp.round(xr * (levels - 1)), 0, levels - 1).astype(jnp.int32)
        ats.append((ii == idx[None, :]).astype(jnp.bfloat16))
    at = jnp.concatenate(ats, axis=0)
    gm = jnp.dot(at, pos_ref[...], preferred_element_type=jnp.float32)
    for g in range(_GI):
        s = jnp.sum(gm[g * levels:(g + 1) * levels] * lw_ref[...], axis=0)
        o_ref[0, g, :] = jnp.where(s > 0, 1.0, -1.0).astype(jnp.float32)


def _tc_part(x, position, level_weight):
    B = x.shape[0]
    P = x.shape[1] * x.shape[2]
    D = position.shape[1]
    levels = level_weight.shape[0]
    x_flat = x.reshape(B // _GI, _GI, P)
    pos_bf = position.astype(jnp.bfloat16)
    dblk = min(_DBLK, D)
    out = pl.pallas_call(
        functools.partial(_tc_body, P=P, levels=levels),
        grid=(D // dblk, B // _GI),
        in_specs=[
            pl.BlockSpec((1, _GI, P), lambda j, b: (b, 0, 0)),
            pl.BlockSpec((P, dblk), lambda j, b: (0, j)),
            pl.BlockSpec((levels, dblk), lambda j, b: (0, j)),
        ],
        out_specs=pl.BlockSpec((1, _GI, dblk), lambda j, b: (b, 0, j)),
        out_shape=jax.ShapeDtypeStruct((B // _GI, _GI, D), jnp.float32),
        compiler_params=pltpu.CompilerParams(
            dimension_semantics=("arbitrary", "arbitrary"),
        ),
    )(x_flat, pos_bf, level_weight)
    return out.reshape(B, D)


def _sc_part(idx, position, lvl, base, B, P, D):
    NW = 32
    DT = 128
    NK = DT // 16
    ND = D // DT
    NB = NW // ND
    B_t = B // NB
    mesh = plsc.VectorSubcoreMesh(core_axis_name="c", subcore_axis_name="s")

    @functools.partial(
        pl.kernel,
        mesh=mesh,
        out_type=jax.ShapeDtypeStruct((B, D), jnp.float32),
        scratch_types=[
            pltpu.VMEM((P, DT), jnp.float32),
            pltpu.VMEM((_BCH, P), jnp.int32),
            pltpu.VMEM((B_t, DT), jnp.float32),
            pltpu.VMEM((DT,), jnp.int32),
            pltpu.VMEM((DT,), jnp.float32),
            pltpu.VMEM((DT,), jnp.float32),
        ],
    )
    def k(idx_hbm, pos_hbm, lvl_hbm, base_hbm, out_hbm,
          pos_v, idx_v, outf_v, lvl_v, base_v, s_v):
        wid = lax.axis_index("s") * 2 + lax.axis_index("c")
        wd = lax.rem(wid, ND)
        wb = wid // ND
        dbase = wd * DT
        bbase = wb * B_t
        pltpu.sync_copy(pos_hbm.at[:, pl.ds(dbase, DT)], pos_v)
        pltpu.sync_copy(lvl_hbm.at[pl.ds(dbase, DT)], lvl_v)
        pltpu.sync_copy(base_hbm.at[pl.ds(dbase, DT)], base_v)

        lvls = [lvl_v[pl.ds(16 * kk, 16)] for kk in range(NK)]
        zero = jnp.zeros((16,), jnp.float32)
        one = jnp.full((16,), 1.0, jnp.float32)
        mone = jnp.full((16,), -1.0, jnp.float32)

        def s_body(p, accs):
            return tuple(
                accs[kk] + pos_v[p, pl.ds(16 * kk, 16)]
                for kk in range(NK)
            )
        svecs = lax.fori_loop(
            0, P, s_body, tuple(zero for _ in range(NK)), unroll=2,
        )
        for kk in range(NK):
            s_v[pl.ds(16 * kk, 16)] = svecs[kk]

        GB = 2
        lm1s = [lvls[kk] - 1 for kk in range(NK)]

        def b_chunk(bc, _):
            pltpu.sync_copy(
                idx_hbm.at[pl.ds(bbase + bc * _BCH, _BCH), :], idx_v)

            def b_group(bg, _):
                def p_body(pc, accs):
                    tv16s = [idx_v[bg * GB + g, pl.ds(pc * 16, 16)]
                             for g in range(GB)]
                    accs = list(accs)
                    for j in range(16):
                        p = pc * 16 + j
                        jv = jnp.full((16,), j, jnp.int32)
                        pks = [pos_v[p, pl.ds(16 * kk, 16)]
                               for kk in range(NK)]
                        for g in range(GB):
                            tv = tv16s[g].at[jv].get(
                                mode="promise_in_bounds")
                            for kk in range(NK):
                                i = g * NK + kk
                                accs[i] = accs[i] + jnp.where(
                                    tv >= lvls[kk], pks[kk], zero)
                    return tuple(accs)

                fvecs = lax.fori_loop(
                    0, P // 16, p_body,
                    tuple(zero for _ in range(GB * NK)),
                )
                for g in range(GB):
                    b = bc * _BCH + bg * GB + g
                    for kk in range(NK):
                        sv = s_v[pl.ds(16 * kk, 16)]
                        bv = base_v[pl.ds(16 * kk, 16)]
                        v = bv * (sv - 2.0 * fvecs[g * NK + kk])
                        outf_v[b, pl.ds(16 * kk, 16)] = jnp.where(
                            v > 0, one, mone)
                return 0

            return lax.fori_loop(0, _BCH // GB, b_group, 0)

        lax.fori_loop(0, B_t // _BCH, b_chunk, 0)
        pltpu.sync_copy(
            outf_v, out_hbm.at[pl.ds(bbase, B_t), pl.ds(dbase, DT)])

    return k(idx, position, lvl, base)


@jax.jit
def kernel(x, position, level_weight):
    B = x.shape[0]
    P = x.shape[1] * x.shape[2]
    D = position.shape[1]
    levels = level_weight.shape[0]

    d_sc = min(_D_SC, D)
    if d_sc == 0:
        return _tc_part(x, position, level_weight)

    flat = x.reshape(B, P)
    idx = jnp.clip(jnp.round(flat * (levels - 1)), 0, levels - 1).astype(jnp.int32)
    base = level_weight[0]
    lvl = jnp.sum((level_weight == base[None, :]).astype(jnp.int32), axis=0)

    sc_out = _sc_part(
        idx, position[:, :d_sc], lvl[:d_sc], base[:d_sc], B, P, d_sc
    )
    if d_sc == D:
        return sc_out
    tc_out = _tc_part(x, position[:, d_sc:], level_weight[:, d_sc:])
    return jnp.concatenate([sc_out, tc_out], axis=1)
